# baseline (speedup 1.0000x reference)
import functools

import jax
import jax.numpy as jnp
from jax import lax
from jax.experimental import pallas as pl
from jax.experimental.pallas import tpu as pltpu
from jax.experimental.pallas import tpu_sc as plsc

N_NODES = 10000
D = 128
N_EDGES = 320000

NC = 2
NS = 16
NW = NC * NS
CK = 128
NCHUNK = N_EDGES // CK
CB = 32
NBATCH = 3
NCHUNK_PAD = 2560
N_PAD = 10240
ROWS_PER_TILE = N_PAD // NS

RB = 1000
GRID = N_NODES // RB


def _sc_mesh():
    return plsc.VectorSubcoreMesh(core_axis_name="c", subcore_axis_name="s",
                                  num_cores=NC)


def _tile_range(w):
    start = jnp.where(w < 24, 80 * w,
                      jnp.where(w < 31, 1920 + 72 * (w - 24), 2424))
    count = jnp.where(w < 24, 80, jnp.where(w < 31, 72, 76))
    return start, count


def _zero_vmem_rows(buf):
    zero16 = jnp.zeros((16,), jnp.float32)

    def zrow(i, carry):
        for j in range(D // 16):
            buf[i, pl.ds(j * 16, 16)] = zero16
        return carry

    lax.fori_loop(0, CK, zrow, 0)


def _deg(dst2):
    @functools.partial(
        pl.kernel,
        mesh=_sc_mesh(),
        out_type=jax.ShapeDtypeStruct((NC, N_PAD, D), jnp.float32),
        scratch_types=[
            pltpu.VMEM((CB, CK), jnp.int32),
            pltpu.VMEM((CK, D), jnp.float32),
            pltpu.VMEM_SHARED((N_PAD, D), jnp.float32),
            pltpu.SemaphoreType.DMA,
        ],
    )
    def deg_kernel(dst_hbm, out_hbm, dbatch, ones_v, acc, sem):
        c = lax.axis_index("c")
        s = lax.axis_index("s")
        w = c * NS + s
        one16 = jnp.ones((16,), jnp.float32)

        _zero_vmem_rows(ones_v)
        row0 = s * ROWS_PER_TILE
        for k in range(ROWS_PER_TILE // CK):
            pltpu.sync_copy(ones_v, acc.at[pl.ds(row0 + k * CK, CK)])

        def orow(i, carry):
            for j in range(D // 16):
                ones_v[i, pl.ds(j * 16, 16)] = one16
            return carry

        lax.fori_loop(0, CK, orow, 0)
        plsc.subcore_barrier()

        start, count = _tile_range(w)

        def batch_body(B, carry):
            t0 = B * CB
            pltpu.sync_copy(dst_hbm.at[pl.ds(start + t0, CB)], dbatch)
            for j in range(CB):
                @pl.when(t0 + j < count)
                def _(j=j):
                    pltpu.async_copy(ones_v, acc.at[dbatch.at[j]], sem,
                                     add=True)
            for j in range(CB):
                @pl.when(t0 + j < count)
                def _(j=j):
                    pltpu.make_async_copy(
                        ones_v, acc.at[dbatch.at[j]], sem).wait()
            return carry

        lax.fori_loop(0, NBATCH, batch_body, 0)

        plsc.subcore_barrier()
        pltpu.sync_copy(
            acc.at[pl.ds(row0, ROWS_PER_TILE)],
            out_hbm.at[c, pl.ds(row0, ROWS_PER_TILE)],
        )

    return deg_kernel(dst2)


def _msg(xs, src2, dst2):
    @functools.partial(
        pl.kernel,
        mesh=_sc_mesh(),
        out_type=jax.ShapeDtypeStruct((NC, N_PAD, D), jnp.float32),
        scratch_types=[
            pltpu.VMEM((CB, CK), jnp.int32),
            pltpu.VMEM((CB, CK), jnp.int32),
            pltpu.VMEM((CK, D), jnp.float32),
            pltpu.VMEM((CK, D), jnp.float32),
            pltpu.VMEM_SHARED((N_PAD, D), jnp.float32),
            pltpu.SemaphoreType.DMA,
            pltpu.SemaphoreType.DMA,
            pltpu.SemaphoreType.DMA,
            pltpu.SemaphoreType.DMA,
        ],
    )
    def msg_kernel(xs_hbm, src_hbm, dst_hbm, out_hbm,
                   sbatch, dbatch, rows0, rows1, acc,
                   sem0, sem1, asem0, asem1):
        c = lax.axis_index("c")
        s = lax.axis_index("s")
        w = c * NS + s
        rows = (rows0, rows1)
        sems = (sem0, sem1)
        asems = (asem0, asem1)

        _zero_vmem_rows(rows0)
        row0 = s * ROWS_PER_TILE
        for k in range(ROWS_PER_TILE // CK):
            pltpu.sync_copy(rows0, acc.at[pl.ds(row0 + k * CK, CK)])
        plsc.subcore_barrier()

        start, count = _tile_range(w)

        def batch_body(B, carry):
            t0 = B * CB
            pltpu.sync_copy(src_hbm.at[pl.ds(start + t0, CB)], sbatch)
            pltpu.sync_copy(dst_hbm.at[pl.ds(start + t0, CB)], dbatch)

            @pl.when(t0 < count)
            def _():
                pltpu.async_copy(xs_hbm.at[sbatch.at[0]], rows0, sem0)

            for j in range(CB):
                b = j % 2

                @pl.when(t0 + j < count)
                def _(j=j, b=b):
                    if j + 1 < CB:
                        @pl.when(t0 + j + 1 < count)
                        def _():
                            if j >= 1:
                                pltpu.make_async_copy(
                                    rows[1 - b], acc.at[dbatch.at[j - 1]],
                                    asems[1 - b]).wait()
                            pltpu.async_copy(
                                xs_hbm.at[sbatch.at[j + 1]],
                                rows[1 - b], sems[1 - b])
                    pltpu.make_async_copy(
                        xs_hbm.at[sbatch.at[j]], rows[b], sems[b]).wait()
                    pltpu.async_copy(rows[b], acc.at[dbatch.at[j]],
                                     asems[b], add=True)

            for j in range(CB):
                if j <= CB - 3:
                    cond = (t0 + j < count) & (t0 + j + 2 >= count)
                else:
                    cond = t0 + j < count

                @pl.when(cond)
                def _(j=j):
                    pltpu.make_async_copy(
                        rows[j % 2], acc.at[dbatch.at[j]],
                        asems[j % 2]).wait()
            return carry

        lax.fori_loop(0, NBATCH, batch_body, 0)

        plsc.subcore_barrier()
        pltpu.sync_copy(
            acc.at[pl.ds(row0, ROWS_PER_TILE)],
            out_hbm.at[c, pl.ds(row0, ROWS_PER_TILE)],
        )

    return msg_kernel(xs, src2, dst2)


def _scales(degp_ref):
    deg = degp_ref[0, :, 0:1] + degp_ref[1, :, 0:1] + 1.0
    dis = lax.rsqrt(deg)
    return dis, 1.0 / deg


def _psum(p_ref):
    return p_ref[0] + p_ref[1]


def _h0_body(x_ref, w_ref, h_ref):
    h_ref[...] = jnp.dot(x_ref[...], w_ref[...],
                         preferred_element_type=jnp.float32)


def _first_body(x_ref, h_ref, degp_ref, b_ref, xs_ref, r_ref):
    dis, inv = _scales(degp_ref)
    h = h_ref[...]
    xs_ref[...] = dis * h
    r_ref[...] = inv * h + b_ref[...] + x_ref[...]


def _mid_body(p_ref, rin_ref, degp_ref, w_ref, b_ref, xs_ref, r_ref):
    dis, inv = _scales(degp_ref)
    xn = dis * _psum(p_ref) + rin_ref[...]
    h = jnp.dot(xn, w_ref[...], preferred_element_type=jnp.float32)
    xs_ref[...] = dis * h
    r_ref[...] = inv * h + b_ref[...] + xn


def _last_body(p_ref, rin_ref, degp_ref, o_ref):
    dis, _ = _scales(degp_ref)
    o_ref[...] = dis * _psum(p_ref) + rin_ref[...]


_ROWS = pl.BlockSpec((RB, D), lambda i: (i, 0))
_DEGS = pl.BlockSpec((NC, RB, D), lambda i: (0, i, 0))
_PART = pl.BlockSpec((NC, RB, D), lambda i: (0, i, 0))
_WSPEC = pl.BlockSpec((D, D), lambda i: (0, 0))
_BSPEC = pl.BlockSpec((1, D), lambda i: (0, 0))
_XSD = jax.ShapeDtypeStruct((N_NODES, D), jnp.float32)


def _tc_h0(x, W):
    return pl.pallas_call(
        _h0_body,
        grid=(GRID,),
        in_specs=[_ROWS, _WSPEC],
        out_specs=_ROWS,
        out_shape=_XSD,
    )(x, W)


def _tc_first(x, h, degp, b):
    return pl.pallas_call(
        _first_body,
        grid=(GRID,),
        in_specs=[_ROWS, _ROWS, _DEGS, _BSPEC],
        out_specs=[_ROWS, _ROWS],
        out_shape=[_XSD, _XSD],
    )(x, h, degp, b)


def _tc_mid(p, rin, degp, W, b):
    return pl.pallas_call(
        _mid_body,
        grid=(GRID,),
        in_specs=[_PART, _ROWS, _DEGS, _WSPEC, _BSPEC],
        out_specs=[_ROWS, _ROWS],
        out_shape=[_XSD, _XSD],
    )(p, rin, degp, W, b)


def _tc_last(p, rin, degp):
    return pl.pallas_call(
        _last_body,
        grid=(GRID,),
        in_specs=[_PART, _ROWS, _DEGS],
        out_specs=_ROWS,
        out_shape=_XSD,
    )(p, rin, degp)


def kernel(x, edge_index, W0, b0, W1, b1, W2, b2):
    src = edge_index[0].astype(jnp.int32)
    dst = edge_index[1].astype(jnp.int32)
    pad = NCHUNK_PAD * CK - N_EDGES
    src2 = jnp.pad(src, (0, pad)).reshape(NCHUNK_PAD, CK)
    dst2 = jnp.pad(dst, (0, pad)).reshape(NCHUNK_PAD, CK)
    degp = _deg(dst2)
    h0 = _tc_h0(x, W0)
    b0r, b1r, b2r = (b.reshape(1, D) for b in (b0, b1, b2))
    xs, r = _tc_first(x, h0, degp, b0r)
    for (W, b) in ((W1, b1r), (W2, b2r)):
        p = _msg(xs, src2, dst2)
        xs, r = _tc_mid(p, r, degp, W, b)
    p = _msg(xs, src2, dst2)
    return _tc_last(p, r, degp)


# speedup vs baseline: 1.1296x; 1.1296x over previous
import functools

import jax
import jax.numpy as jnp
from jax import lax
from jax.experimental import pallas as pl
from jax.experimental.pallas import tpu as pltpu
from jax.experimental.pallas import tpu_sc as plsc

N_NODES = 10000
D = 128
N_EDGES = 320000

NC = 2
NS = 16
NW = NC * NS
CK = 128
NCHUNK = N_EDGES // CK
CB = 32
NBATCH = 3
NCHUNK_PAD = 2560
N_PAD = 10240
ROWS_PER_TILE = N_PAD // NS

RB = 1000
GRID = N_NODES // RB


def _sc_mesh():
    return plsc.VectorSubcoreMesh(core_axis_name="c", subcore_axis_name="s",
                                  num_cores=NC)


def _tile_range(w):
    start = jnp.where(w < 24, 80 * w,
                      jnp.where(w < 31, 1920 + 72 * (w - 24), 2424))
    count = jnp.where(w < 24, 80, jnp.where(w < 31, 72, 76))
    return start, count


def _zero_vmem_rows(buf):
    zero16 = jnp.zeros((16,), jnp.float32)

    def zrow(i, carry):
        for j in range(D // 16):
            buf[i, pl.ds(j * 16, 16)] = zero16
        return carry

    lax.fori_loop(0, CK, zrow, 0)


HR = N_PAD // D


def _deg(dst2):
    @functools.partial(
        pl.kernel,
        mesh=_sc_mesh(),
        out_type=jax.ShapeDtypeStruct((NC, HR, D), jnp.float32),
        scratch_types=[
            pltpu.VMEM((CB, CK), jnp.int32),
            pltpu.VMEM((HR, D), jnp.float32),
            pltpu.VMEM((HR,), jnp.int32),
            pltpu.VMEM_SHARED((HR, D), jnp.float32),
        ],
        compiler_params=pltpu.CompilerParams(needs_layout_passes=False),
    )
    def deg_kernel(dst_hbm, out_hbm, dbatch, hist, riota, accs):
        c = lax.axis_index("c")
        s = lax.axis_index("s")
        w = c * NS + s
        one16 = jnp.ones((16,), jnp.float32)
        zero16 = jnp.zeros((16,), jnp.float32)
        i16 = lax.iota(jnp.int32, 16)

        def zrow(i, carry):
            for j in range(D // 16):
                hist[i, pl.ds(j * 16, 16)] = zero16
            return carry

        lax.fori_loop(0, HR, zrow, 0)
        for k in range(HR // 16):
            riota[pl.ds(k * 16, 16)] = i16 + (k * 16)

        @pl.when(s == 0)
        def _():
            pltpu.sync_copy(hist, accs)

        plsc.subcore_barrier()

        start, count = _tile_range(w)

        def batch_body(B, carry):
            t0 = B * CB
            pltpu.sync_copy(dst_hbm.at[pl.ds(start + t0, CB)], dbatch)
            for j in range(CB):
                @pl.when(t0 + j < count)
                def _(j=j):
                    for k in range(CK // 16):
                        idx = dbatch[j, pl.ds(k * 16, 16)]
                        hi = lax.shift_right_logical(idx, 7)
                        lo = lax.bitwise_and(idx, 127)
                        plsc.addupdate_scatter(hist, [hi, lo], one16)
            return carry

        lax.fori_loop(0, NBATCH, batch_body, 0)

        pltpu.sync_copy(hist, accs.at[riota], add=True)
        plsc.subcore_barrier()

        @pl.when(s == 0)
        def _():
            pltpu.sync_copy(accs, out_hbm.at[c])

    return deg_kernel(dst2)


def _msg(xs, src2, dst2):
    @functools.partial(
        pl.kernel,
        mesh=_sc_mesh(),
        out_type=jax.ShapeDtypeStruct((NC, N_PAD, D), jnp.float32),
        scratch_types=[
            pltpu.VMEM((CB, CK), jnp.int32),
            pltpu.VMEM((CB, CK), jnp.int32),
            pltpu.VMEM((CK, D), jnp.float32),
            pltpu.VMEM((CK, D), jnp.float32),
            pltpu.VMEM_SHARED((N_PAD, D), jnp.float32),
            pltpu.SemaphoreType.DMA,
            pltpu.SemaphoreType.DMA,
            pltpu.SemaphoreType.DMA,
            pltpu.SemaphoreType.DMA,
        ],
    )
    def msg_kernel(xs_hbm, src_hbm, dst_hbm, out_hbm,
                   sbatch, dbatch, rows0, rows1, acc,
                   sem0, sem1, asem0, asem1):
        c = lax.axis_index("c")
        s = lax.axis_index("s")
        w = c * NS + s
        rows = (rows0, rows1)
        sems = (sem0, sem1)
        asems = (asem0, asem1)

        _zero_vmem_rows(rows0)
        row0 = s * ROWS_PER_TILE
        for k in range(ROWS_PER_TILE // CK):
            pltpu.sync_copy(rows0, acc.at[pl.ds(row0 + k * CK, CK)])
        plsc.subcore_barrier()

        start, count = _tile_range(w)

        def batch_body(B, carry):
            t0 = B * CB
            pltpu.sync_copy(src_hbm.at[pl.ds(start + t0, CB)], sbatch)
            pltpu.sync_copy(dst_hbm.at[pl.ds(start + t0, CB)], dbatch)

            @pl.when(t0 < count)
            def _():
                pltpu.async_copy(xs_hbm.at[sbatch.at[0]], rows0, sem0)

            for j in range(CB):
                b = j % 2

                @pl.when(t0 + j < count)
                def _(j=j, b=b):
                    if j + 1 < CB:
                        @pl.when(t0 + j + 1 < count)
                        def _():
                            if j >= 1:
                                pltpu.make_async_copy(
                                    rows[1 - b], acc.at[dbatch.at[j - 1]],
                                    asems[1 - b]).wait()
                            pltpu.async_copy(
                                xs_hbm.at[sbatch.at[j + 1]],
                                rows[1 - b], sems[1 - b])
                    pltpu.make_async_copy(
                        xs_hbm.at[sbatch.at[j]], rows[b], sems[b]).wait()
                    pltpu.async_copy(rows[b], acc.at[dbatch.at[j]],
                                     asems[b], add=True)

            for j in range(CB):
                if j <= CB - 3:
                    cond = (t0 + j < count) & (t0 + j + 2 >= count)
                else:
                    cond = t0 + j < count

                @pl.when(cond)
                def _(j=j):
                    pltpu.make_async_copy(
                        rows[j % 2], acc.at[dbatch.at[j]],
                        asems[j % 2]).wait()
            return carry

        lax.fori_loop(0, NBATCH, batch_body, 0)

        plsc.subcore_barrier()
        pltpu.sync_copy(
            acc.at[pl.ds(row0, ROWS_PER_TILE)],
            out_hbm.at[c, pl.ds(row0, ROWS_PER_TILE)],
        )

    return msg_kernel(xs, src2, dst2)


def _scales(degc_ref):
    deg = degc_ref[...] + 1.0
    dis = lax.rsqrt(deg)
    return dis, 1.0 / deg


def _psum(p_ref):
    return p_ref[0] + p_ref[1]


def _h0_body(x_ref, w_ref, h_ref):
    h_ref[...] = jnp.dot(x_ref[...], w_ref[...],
                         preferred_element_type=jnp.float32)


def _first_body(x_ref, h_ref, degp_ref, b_ref, xs_ref, r_ref):
    dis, inv = _scales(degp_ref)
    h = h_ref[...]
    xs_ref[...] = dis * h
    r_ref[...] = inv * h + b_ref[...] + x_ref[...]


def _mid_body(p_ref, rin_ref, degp_ref, w_ref, b_ref, xs_ref, r_ref):
    dis, inv = _scales(degp_ref)
    xn = dis * _psum(p_ref) + rin_ref[...]
    h = jnp.dot(xn, w_ref[...], preferred_element_type=jnp.float32)
    xs_ref[...] = dis * h
    r_ref[...] = inv * h + b_ref[...] + xn


def _last_body(p_ref, rin_ref, degp_ref, o_ref):
    dis, _ = _scales(degp_ref)
    o_ref[...] = dis * _psum(p_ref) + rin_ref[...]


_ROWS = pl.BlockSpec((RB, D), lambda i: (i, 0))
_DEGS = pl.BlockSpec((RB, 1), lambda i: (i, 0))
_PART = pl.BlockSpec((NC, RB, D), lambda i: (0, i, 0))
_WSPEC = pl.BlockSpec((D, D), lambda i: (0, 0))
_BSPEC = pl.BlockSpec((1, D), lambda i: (0, 0))
_XSD = jax.ShapeDtypeStruct((N_NODES, D), jnp.float32)


def _tc_h0(x, W):
    return pl.pallas_call(
        _h0_body,
        grid=(GRID,),
        in_specs=[_ROWS, _WSPEC],
        out_specs=_ROWS,
        out_shape=_XSD,
    )(x, W)


def _tc_first(x, h, degp, b):
    return pl.pallas_call(
        _first_body,
        grid=(GRID,),
        in_specs=[_ROWS, _ROWS, _DEGS, _BSPEC],
        out_specs=[_ROWS, _ROWS],
        out_shape=[_XSD, _XSD],
    )(x, h, degp, b)


def _tc_mid(p, rin, degp, W, b):
    return pl.pallas_call(
        _mid_body,
        grid=(GRID,),
        in_specs=[_PART, _ROWS, _DEGS, _WSPEC, _BSPEC],
        out_specs=[_ROWS, _ROWS],
        out_shape=[_XSD, _XSD],
    )(p, rin, degp, W, b)


def _tc_last(p, rin, degp):
    return pl.pallas_call(
        _last_body,
        grid=(GRID,),
        in_specs=[_PART, _ROWS, _DEGS],
        out_specs=_ROWS,
        out_shape=_XSD,
    )(p, rin, degp)


def kernel(x, edge_index, W0, b0, W1, b1, W2, b2):
    src = edge_index[0].astype(jnp.int32)
    dst = edge_index[1].astype(jnp.int32)
    pad = NCHUNK_PAD * CK - N_EDGES
    src2 = jnp.pad(src, (0, pad)).reshape(NCHUNK_PAD, CK)
    dst2 = jnp.pad(dst, (0, pad)).reshape(NCHUNK_PAD, CK)
    degp = _deg(dst2)
    h0 = _tc_h0(x, W0)
    degc = (degp[0] + degp[1]).reshape(N_PAD, 1)[:N_NODES]
    b0r, b1r, b2r = (b.reshape(1, D) for b in (b0, b1, b2))
    xs, r = _tc_first(x, h0, degc, b0r)
    for (W, b) in ((W1, b1r), (W2, b2r)):
        p = _msg(xs, src2, dst2)
        xs, r = _tc_mid(p, r, degc, W, b)
    p = _msg(xs, src2, dst2)
    return _tc_last(p, r, degc)


# speedup vs baseline: 1.1563x; 1.0236x over previous
import functools

import jax
import jax.numpy as jnp
from jax import lax
from jax.experimental import pallas as pl
from jax.experimental.pallas import tpu as pltpu
from jax.experimental.pallas import tpu_sc as plsc

N_NODES = 10000
D = 128
N_EDGES = 320000

NC = 2
NS = 16
NW = NC * NS
CK = 128
NCHUNK = N_EDGES // CK
CB = 32
NBATCH = 3
NCHUNK_PAD = 2560
N_PAD = 10240
ROWS_PER_TILE = N_PAD // NS

RB = 1000
GRID = N_NODES // RB


def _sc_mesh():
    return plsc.VectorSubcoreMesh(core_axis_name="c", subcore_axis_name="s",
                                  num_cores=NC)


def _tile_range(w):
    start = jnp.where(w < 24, 80 * w,
                      jnp.where(w < 31, 1920 + 72 * (w - 24), 2424))
    count = jnp.where(w < 24, 80, jnp.where(w < 31, 72, 76))
    return start, count


def _zero_vmem_rows(buf):
    zero16 = jnp.zeros((16,), jnp.float32)

    def zrow(i, carry):
        for j in range(D // 16):
            buf[i, pl.ds(j * 16, 16)] = zero16
        return carry

    lax.fori_loop(0, CK, zrow, 0)


HR = N_PAD // D


def _deg(dst2):
    @functools.partial(
        pl.kernel,
        mesh=_sc_mesh(),
        out_type=jax.ShapeDtypeStruct((NC, HR, D), jnp.float32),
        scratch_types=[
            pltpu.VMEM((CB, CK), jnp.int32),
            pltpu.VMEM((HR, D), jnp.float32),
            pltpu.VMEM((HR,), jnp.int32),
            pltpu.VMEM_SHARED((HR, D), jnp.float32),
        ],
        compiler_params=pltpu.CompilerParams(needs_layout_passes=False),
    )
    def deg_kernel(dst_hbm, out_hbm, dbatch, hist, riota, accs):
        c = lax.axis_index("c")
        s = lax.axis_index("s")
        w = c * NS + s
        one16 = jnp.ones((16,), jnp.float32)
        zero16 = jnp.zeros((16,), jnp.float32)
        i16 = lax.iota(jnp.int32, 16)

        def zrow(i, carry):
            for j in range(D // 16):
                hist[i, pl.ds(j * 16, 16)] = zero16
            return carry

        lax.fori_loop(0, HR, zrow, 0)
        for k in range(HR // 16):
            riota[pl.ds(k * 16, 16)] = i16 + (k * 16)

        @pl.when(s == 0)
        def _():
            pltpu.sync_copy(hist, accs)

        plsc.subcore_barrier()

        start, count = _tile_range(w)

        def batch_body(B, carry):
            t0 = B * CB
            pltpu.sync_copy(dst_hbm.at[pl.ds(start + t0, CB)], dbatch)
            for j in range(CB):
                @pl.when(t0 + j < count)
                def _(j=j):
                    for k in range(CK // 16):
                        idx = dbatch[j, pl.ds(k * 16, 16)]
                        hi = lax.shift_right_logical(idx, 7)
                        lo = lax.bitwise_and(idx, 127)
                        plsc.addupdate_scatter(hist, [hi, lo], one16)
            return carry

        lax.fori_loop(0, NBATCH, batch_body, 0)

        pltpu.sync_copy(hist, accs.at[riota], add=True)
        plsc.subcore_barrier()

        @pl.when(s == 0)
        def _():
            pltpu.sync_copy(accs, out_hbm.at[c])

    return deg_kernel(dst2)


def _msg(xs, src2, dst2):
    @functools.partial(
        pl.kernel,
        mesh=_sc_mesh(),
        out_type=jax.ShapeDtypeStruct((NC, N_PAD, D), jnp.float32),
        scratch_types=[
            pltpu.VMEM((CB, CK), jnp.int32),
            pltpu.VMEM((CB, CK), jnp.int32),
            pltpu.VMEM((CK, D), jnp.float32),
            pltpu.VMEM((CK, D), jnp.float32),
            pltpu.VMEM_SHARED((N_PAD, D), jnp.float32),
            pltpu.SemaphoreType.DMA,
            pltpu.SemaphoreType.DMA,
            pltpu.SemaphoreType.DMA,
            pltpu.SemaphoreType.DMA,
            pltpu.SemaphoreType.DMA,
            pltpu.SemaphoreType.DMA,
        ],
    )
    def msg_kernel(xs_hbm, src_hbm, dst_hbm, out_hbm,
                   sbatch, dbatch, rows0, rows1, acc,
                   sem0, sem1, asem0, asem1, isem0, isem1):
        c = lax.axis_index("c")
        s = lax.axis_index("s")
        w = c * NS + s
        rows = (rows0, rows1)
        sems = (sem0, sem1)
        asems = (asem0, asem1)

        start, count = _tile_range(w)
        pltpu.async_copy(src_hbm.at[pl.ds(start, CB)], sbatch, isem0)
        pltpu.async_copy(dst_hbm.at[pl.ds(start, CB)], dbatch, isem1)

        _zero_vmem_rows(rows0)
        row0 = s * ROWS_PER_TILE
        for k in range(ROWS_PER_TILE // CK):
            pltpu.sync_copy(rows0, acc.at[pl.ds(row0 + k * CK, CK)])
        plsc.subcore_barrier()

        def batch_body(B, carry):
            t0 = B * CB

            @pl.when(B > 0)
            def _():
                pltpu.async_copy(src_hbm.at[pl.ds(start + t0, CB)], sbatch,
                                 isem0)
                pltpu.async_copy(dst_hbm.at[pl.ds(start + t0, CB)], dbatch,
                                 isem1)

            pltpu.make_async_copy(src_hbm.at[pl.ds(start + t0, CB)], sbatch,
                                  isem0).wait()
            pltpu.make_async_copy(dst_hbm.at[pl.ds(start + t0, CB)], dbatch,
                                  isem1).wait()

            @pl.when(t0 < count)
            def _():
                pltpu.async_copy(xs_hbm.at[sbatch.at[0]], rows0, sem0)

            for j in range(CB):
                b = j % 2

                @pl.when(t0 + j < count)
                def _(j=j, b=b):
                    if j + 1 < CB:
                        @pl.when(t0 + j + 1 < count)
                        def _():
                            if j >= 1:
                                pltpu.make_async_copy(
                                    rows[1 - b], acc.at[dbatch.at[j - 1]],
                                    asems[1 - b]).wait()
                            pltpu.async_copy(
                                xs_hbm.at[sbatch.at[j + 1]],
                                rows[1 - b], sems[1 - b])
                    pltpu.make_async_copy(
                        xs_hbm.at[sbatch.at[j]], rows[b], sems[b]).wait()
                    pltpu.async_copy(rows[b], acc.at[dbatch.at[j]],
                                     asems[b], add=True)

            for j in range(CB):
                if j <= CB - 3:
                    cond = (t0 + j < count) & (t0 + j + 2 >= count)
                else:
                    cond = t0 + j < count

                @pl.when(cond)
                def _(j=j):
                    pltpu.make_async_copy(
                        rows[j % 2], acc.at[dbatch.at[j]],
                        asems[j % 2]).wait()
            return carry

        lax.fori_loop(0, NBATCH, batch_body, 0)

        plsc.subcore_barrier()
        pltpu.sync_copy(
            acc.at[pl.ds(row0, ROWS_PER_TILE)],
            out_hbm.at[c, pl.ds(row0, ROWS_PER_TILE)],
        )

    return msg_kernel(xs, src2, dst2)


def _scales(degc_ref):
    deg = degc_ref[...] + 1.0
    dis = lax.rsqrt(deg)
    return dis, 1.0 / deg


def _psum(p_ref):
    return p_ref[0] + p_ref[1]


def _h0_body(x_ref, w_ref, h_ref):
    h_ref[...] = jnp.dot(x_ref[...], w_ref[...],
                         preferred_element_type=jnp.float32)


def _first_body(x_ref, h_ref, degp_ref, b_ref, xs_ref, r_ref):
    dis, inv = _scales(degp_ref)
    h = h_ref[...]
    xs_ref[...] = dis * h
    r_ref[...] = inv * h + b_ref[...] + x_ref[...]


def _mid_body(p_ref, rin_ref, degp_ref, w_ref, b_ref, xs_ref, r_ref):
    dis, inv = _scales(degp_ref)
    xn = dis * _psum(p_ref) + rin_ref[...]
    h = jnp.dot(xn, w_ref[...], preferred_element_type=jnp.float32)
    xs_ref[...] = dis * h
    r_ref[...] = inv * h + b_ref[...] + xn


def _last_body(p_ref, rin_ref, degp_ref, o_ref):
    dis, _ = _scales(degp_ref)
    o_ref[...] = dis * _psum(p_ref) + rin_ref[...]


_ROWS = pl.BlockSpec((RB, D), lambda i: (i, 0))
_DEGS = pl.BlockSpec((RB, 1), lambda i: (i, 0))
_PART = pl.BlockSpec((NC, RB, D), lambda i: (0, i, 0))
_WSPEC = pl.BlockSpec((D, D), lambda i: (0, 0))
_BSPEC = pl.BlockSpec((1, D), lambda i: (0, 0))
_XSD = jax.ShapeDtypeStruct((N_NODES, D), jnp.float32)


def _tc_h0(x, W):
    return pl.pallas_call(
        _h0_body,
        grid=(GRID,),
        in_specs=[_ROWS, _WSPEC],
        out_specs=_ROWS,
        out_shape=_XSD,
    )(x, W)


def _tc_first(x, h, degp, b):
    return pl.pallas_call(
        _first_body,
        grid=(GRID,),
        in_specs=[_ROWS, _ROWS, _DEGS, _BSPEC],
        out_specs=[_ROWS, _ROWS],
        out_shape=[_XSD, _XSD],
    )(x, h, degp, b)


def _tc_mid(p, rin, degp, W, b):
    return pl.pallas_call(
        _mid_body,
        grid=(GRID,),
        in_specs=[_PART, _ROWS, _DEGS, _WSPEC, _BSPEC],
        out_specs=[_ROWS, _ROWS],
        out_shape=[_XSD, _XSD],
    )(p, rin, degp, W, b)


def _tc_last(p, rin, degp):
    return pl.pallas_call(
        _last_body,
        grid=(GRID,),
        in_specs=[_PART, _ROWS, _DEGS],
        out_specs=_ROWS,
        out_shape=_XSD,
    )(p, rin, degp)


def kernel(x, edge_index, W0, b0, W1, b1, W2, b2):
    src = edge_index[0].astype(jnp.int32)
    dst = edge_index[1].astype(jnp.int32)
    pad = NCHUNK_PAD * CK - N_EDGES
    src2 = jnp.pad(src, (0, pad)).reshape(NCHUNK_PAD, CK)
    dst2 = jnp.pad(dst, (0, pad)).reshape(NCHUNK_PAD, CK)
    degp = _deg(dst2)
    h0 = _tc_h0(x, W0)
    degc = (degp[0] + degp[1]).reshape(N_PAD, 1)[:N_NODES]
    b0r, b1r, b2r = (b.reshape(1, D) for b in (b0, b1, b2))
    xs, r = _tc_first(x, h0, degc, b0r)
    for (W, b) in ((W1, b1r), (W2, b2r)):
        p = _msg(xs, src2, dst2)
        xs, r = _tc_mid(p, r, degc, W, b)
    p = _msg(xs, src2, dst2)
    return _tc_last(p, r, degc)
